# baseline (speedup 1.0000x reference)
import functools

import jax
import jax.numpy as jnp
from jax import lax
from jax.experimental import pallas as pl
from jax.experimental.pallas import tpu as pltpu
from jax.experimental.pallas import tpu_sc as plsc

N_USERS = 25000
N_NODES = 50000
D = 64
E = 800000
B = 4096

NC = 2
NS = 16
NW = NC * NS
HALF = N_NODES // NC
TROWS = 1568
HP = NS * TROWS
NP = NC * HP
PAD = HP - HALF

SUB = 112
NSUB = 2
CHUNK = SUB * NSUB
EP = 802816
NROWS_IN = EP // SUB
WROWS = NROWS_IN // NW
NREG = 2 * NW
RROWS = WROWS
STG = 240

PB = B // NW

_mesh = plsc.VectorSubcoreMesh(
    core_axis_name="c", subcore_axis_name="s", num_cores=NC, num_subcores=NS)

_params = pltpu.CompilerParams(
    use_tc_tiling_on_sc=False, needs_layout_passes=False)


@functools.partial(
    pl.kernel,
    out_type=(jax.ShapeDtypeStruct((NREG * RROWS, SUB), jnp.int32),
              jax.ShapeDtypeStruct((NREG * RROWS, SUB), jnp.int32),
              jax.ShapeDtypeStruct((NREG * RROWS, SUB), jnp.int32),
              jax.ShapeDtypeStruct((NREG * 16,), jnp.int32)),
    mesh=_mesh,
    compiler_params=_params,
    scratch_types=[
        pltpu.VMEM((NSUB, SUB), jnp.int32),
        pltpu.VMEM((NSUB, SUB), jnp.int32),
        pltpu.VMEM((NSUB, SUB), jnp.int32),
        pltpu.VMEM((1, STG), jnp.int32),
        pltpu.VMEM((1, STG), jnp.int32),
        pltpu.VMEM((1, STG), jnp.int32),
        pltpu.VMEM((1, STG), jnp.int32),
        pltpu.VMEM((1, STG), jnp.int32),
        pltpu.VMEM((1, STG), jnp.int32),
        pltpu.VMEM((16,), jnp.int32),
    ],
)
def _partition(src_in, dst_in, w_in, src_out, dst_out, w_out, cnt_out,
               ins, ind, inw, sa, da, wa, sb, db, wb, cntv):
    c = lax.axis_index("c")
    s = lax.axis_index("s")
    w = s * NC + c
    in0 = w * WROWS
    rb_a = (w * 2 + 0) * RROWS
    rb_b = (w * 2 + 1) * RROWS

    zero16 = jnp.zeros((16,), jnp.int32)
    iota16 = lax.iota(jnp.int32, 16)

    for buf in (sa, da, wa, sb, db, wb):
        for g in range(STG // 16):
            buf[0, pl.ds(g * 16, 16)] = zero16

    def flush(stg3, rb, orow):
        for stg, outp in zip(stg3, (src_out, dst_out, w_out)):
            pltpu.sync_copy(stg.at[pl.ds(0, 1), pl.ds(0, SUB)],
                            outp.at[pl.ds(rb + orow, 1)])
        for stg in stg3:
            for g in range((STG - SUB) // 16):
                stg[0, pl.ds(g * 16, 16)] = stg[0, pl.ds(SUB + g * 16, 16)]

    def chunk(k, carry):
        pos_a, orow_a, pos_b, orow_b = carry
        pltpu.sync_copy(src_in.at[pl.ds(in0 + k * NSUB, NSUB)], ins)
        pltpu.sync_copy(dst_in.at[pl.ds(in0 + k * NSUB, NSUB)], ind)
        pltpu.sync_copy(w_in.at[pl.ds(in0 + k * NSUB, NSUB)], inw)
        for r in range(NSUB):
            for g in range(SUB // 16):
                sl = pl.ds(g * 16, 16)
                s16 = ins[r, sl]
                d16 = ind[r, sl]
                w16 = inw[r, sl]
                m_a = d16 < HALF
                plsc.store_compressed(sa.at[0, pl.ds(pos_a, 16)], s16, mask=m_a)
                plsc.store_compressed(da.at[0, pl.ds(pos_a, 16)], d16, mask=m_a)
                plsc.store_compressed(wa.at[0, pl.ds(pos_a, 16)], w16, mask=m_a)
                m_b = jnp.logical_not(m_a)
                d16b = d16 - HALF
                plsc.store_compressed(sb.at[0, pl.ds(pos_b, 16)], s16, mask=m_b)
                plsc.store_compressed(db.at[0, pl.ds(pos_b, 16)], d16b, mask=m_b)
                plsc.store_compressed(wb.at[0, pl.ds(pos_b, 16)], w16, mask=m_b)
                n_a = lax.reduce_max(
                    plsc.all_reduce_population_count(m_a), axes=(0,))
                pos_a = pos_a + n_a
                pos_b = pos_b + (16 - n_a)
            do_a = pos_a >= SUB

            @pl.when(do_a)
            def _():
                flush((sa, da, wa), rb_a, orow_a)

            pos_a = jnp.where(do_a, pos_a - SUB, pos_a)
            orow_a = jnp.where(do_a, orow_a + 1, orow_a)

            do_b = pos_b >= SUB

            @pl.when(do_b)
            def _():
                flush((sb, db, wb), rb_b, orow_b)

            pos_b = jnp.where(do_b, pos_b - SUB, pos_b)
            orow_b = jnp.where(do_b, orow_b + 1, orow_b)
        return pos_a, orow_a, pos_b, orow_b

    z = jnp.int32(0)
    pos_a, orow_a, pos_b, orow_b = lax.fori_loop(
        0, WROWS // NSUB, chunk, (z, z, z, z))

    def finish(stg3, rb, pos, orow):
        stg_w = stg3[2]
        for g in range(SUB // 16):
            sl = pl.ds(g * 16, 16)
            keep = (iota16 + g * 16) < pos
            stg_w[0, sl] = jnp.where(keep, stg_w[0, sl], 0)

        @pl.when(pos > 0)
        def _():
            flush(stg3, rb, orow)

        orow = jnp.where(pos > 0, orow + 1, orow)
        for g in range(STG // 16):
            stg_w[0, pl.ds(g * 16, 16)] = zero16
        for _ in range(3):
            do = lax.rem(orow, jnp.int32(4)) != 0

            @pl.when(do)
            def _():
                flush(stg3, rb, orow)

            orow = jnp.where(do, orow + 1, orow)
        return orow

    orow_a = finish((sa, da, wa), rb_a, pos_a, orow_a)
    orow_b = finish((sb, db, wb), rb_b, pos_b, orow_b)

    cntv[pl.ds(0, 16)] = jnp.full((16,), 1, jnp.int32) * (orow_a // NSUB)
    pltpu.sync_copy(cntv, cnt_out.at[pl.ds((w * 2 + 0) * 16, 16)])
    cntv[pl.ds(0, 16)] = jnp.full((16,), 1, jnp.int32) * (orow_b // NSUB)
    pltpu.sync_copy(cntv, cnt_out.at[pl.ds((w * 2 + 1) * 16, 16)])


@functools.partial(
    pl.kernel,
    out_type=jax.ShapeDtypeStruct((NP, D), jnp.float32),
    mesh=_mesh,
    compiler_params=_params,
    scratch_types=[
        pltpu.VMEM((NSUB, SUB), jnp.int32),
        pltpu.VMEM((NSUB, SUB), jnp.int32),
        pltpu.VMEM((NSUB, SUB), jnp.int32),
        pltpu.VMEM((NSUB, SUB), jnp.int32),
        pltpu.VMEM((NSUB, SUB), jnp.int32),
        pltpu.VMEM((NSUB, SUB), jnp.int32),
        pltpu.VMEM((CHUNK, D), jnp.float32),
        pltpu.VMEM((CHUNK, D), jnp.float32),
        pltpu.VMEM((NSUB, SUB), jnp.int32),
        pltpu.VMEM((NSUB, SUB), jnp.int32),
        pltpu.VMEM((16,), jnp.int32),
        pltpu.VMEM_SHARED((HP, D), jnp.float32),
        pltpu.SemaphoreType.DMA,
        pltpu.SemaphoreType.DMA,
        pltpu.SemaphoreType.DMA,
        pltpu.SemaphoreType.DMA,
        pltpu.SemaphoreType.DMA,
        pltpu.SemaphoreType.DMA,
    ],
)
def _layer(emb, src_p, dst_p, w_p, cnt_p, zeros_hbm, out,
           ea, da, wa, eb, db, wb, ra, rb_, dca, dcb, cntv, acc,
           semea, semeb, semga, semgb, semsa, semsb):
    c = lax.axis_index("c")
    s = lax.axis_index("s")
    r0 = s * TROWS

    sets = ((ea, da, wa, ra, dca, semea, semga, semsa),
            (eb, db, wb, rb_, dcb, semeb, semgb, semsb))

    pltpu.sync_copy(zeros_hbm, acc.at[pl.ds(r0, TROWS)])
    plsc.subcore_barrier()

    def region(i):
        reg = (2 * s + i) * 2 + c
        rbase = reg * RROWS

        pltpu.sync_copy(cnt_p.at[pl.ds(reg * 16, 16)], cntv)
        kd = lax.reduce_max(cntv[pl.ds(0, 16)], axes=(0,))

        def issue_edata(k, st):
            es, ds_, ws, _, _, seme, _, _ = st
            row = rbase + k * NSUB
            pltpu.async_copy(src_p.at[pl.ds(row, NSUB)], es, seme)
            pltpu.async_copy(dst_p.at[pl.ds(row, NSUB)], ds_, seme)
            pltpu.async_copy(w_p.at[pl.ds(row, NSUB)], ws, seme)

        def drain_edata(st):
            es, ds_, ws, _, _, seme, _, _ = st
            for buf in (es, ds_, ws):
                pltpu.make_async_copy(
                    src_p.at[pl.ds(0, NSUB)], buf, seme).wait()

        def issue_gather(st):
            es, _, _, r, _, _, semg, _ = st
            for j in range(NSUB):
                pltpu.async_copy(emb.at[es.at[j]],
                                 r.at[pl.ds(j * SUB, SUB)], semg)

        def drain_rows(st, which):
            _, _, _, r, _, _, semg, sems = st
            sem = semg if which == "g" else sems
            pltpu.make_async_copy(out.at[pl.ds(0, CHUNK)], r, sem).wait()

        def issue_scatter(st):
            _, _, _, r, dc, _, _, sems = st
            for j in range(NSUB):
                pltpu.async_copy(r.at[pl.ds(j * SUB, SUB)],
                                 acc.at[dc.at[j]], sems, add=True)

        def compute(st):
            _, ds_, ws, r, dc, _, _, _ = st

            for jj in range(NSUB):
                for g in range(SUB // 16):
                    sl = pl.ds(g * 16, 16)
                    dc[jj, sl] = ds_[jj, sl]

            def row_body(jj, rc):
                for cc in range(SUB // 16):
                    sl = pl.ds(cc * 16, 16)
                    w16 = plsc.bitcast(ws[jj, sl], jnp.float32)
                    e0 = jj * SUB + cc * 16
                    for i2 in range(16):
                        wsp = w16.at[jnp.full((16,), i2, jnp.int32)].get(
                            mode="promise_in_bounds")
                        for j in range(D // 16):
                            sj = pl.ds(j * 16, 16)
                            r[e0 + i2, sj] = r[e0 + i2, sj] * wsp
                return rc

            lax.fori_loop(0, NSUB, row_body, 0)

        @pl.when(kd > 0)
        def _():
            issue_edata(0, sets[0])
            drain_edata(sets[0])
            issue_gather(sets[0])

        @pl.when(kd > 1)
        def _():
            issue_edata(1, sets[1])

        def step(k, cur, nxt):
            @pl.when(k + 1 < kd)
            def _():
                drain_edata(nxt)

            @pl.when(k >= 1)
            def _():
                drain_rows(nxt, "s")

            @pl.when(k + 1 < kd)
            def _():
                issue_gather(nxt)

            drain_rows(cur, "g")
            compute(cur)
            issue_scatter(cur)

            @pl.when(k + 2 < kd)
            def _():
                issue_edata(k + 2, cur)

        def chunk_pair(kk, carry):
            step(2 * kk, sets[0], sets[1])
            step(2 * kk + 1, sets[1], sets[0])
            return carry

        lax.fori_loop(0, kd // 2, chunk_pair, 0)

        @pl.when(kd > 0)
        def _():
            drain_rows(sets[1], "s")

    region(0)
    region(1)
    plsc.subcore_barrier()

    pltpu.sync_copy(acc.at[pl.ds(r0, TROWS)],
                    out.at[pl.ds(c * HP + r0, TROWS)])


@functools.partial(
    pl.kernel,
    out_type=(jax.ShapeDtypeStruct((B,), jnp.float32),
              jax.ShapeDtypeStruct((B,), jnp.float32)),
    mesh=_mesh,
    compiler_params=_params,
    scratch_types=[
        pltpu.VMEM((PB,), jnp.int32),
        pltpu.VMEM((PB, D), jnp.float32),
        pltpu.VMEM((PB, D), jnp.float32),
        pltpu.VMEM((PB, D), jnp.float32),
        pltpu.VMEM((PB,), jnp.float32),
        pltpu.SemaphoreType.DMA,
    ],
)
def _score(t0, t1, t2, t3, users_h, pos_h, neg_h, pos_out, neg_out,
           idxv, ua, pb, tmp, scv, sem):
    c = lax.axis_index("c")
    s = lax.axis_index("s")
    wid = s * NC + c
    base = wid * PB

    def gather_sum(dst_buf):
        pltpu.async_copy(t0.at[idxv], dst_buf, sem).wait()
        for t in (t1, t2, t3):
            pltpu.async_copy(t.at[idxv], tmp, sem).wait()

            def add_body(r, rc):
                for j in range(D // 16):
                    sj = pl.ds(j * 16, 16)
                    dst_buf[r, sj] = dst_buf[r, sj] + tmp[r, sj]
                return rc

            lax.fori_loop(0, PB, add_body, 0)

    lane0 = lax.iota(jnp.int32, 16) == 0

    def dots():
        def dot_body(e, rc):
            acc16 = ua[e, pl.ds(0, 16)] * pb[e, pl.ds(0, 16)]
            for j in range(1, D // 16):
                sj = pl.ds(j * 16, 16)
                acc16 = acc16 + ua[e, sj] * pb[e, sj]
            sc = jnp.sum(acc16) * jnp.float32(1.0 / 16.0)
            plsc.store_scatter(scv, [jnp.full((16,), e, jnp.int32)],
                               jnp.full((16,), sc, jnp.float32), mask=lane0)
            return rc

        lax.fori_loop(0, PB, dot_body, 0)

    pltpu.sync_copy(users_h.at[pl.ds(base, PB)], idxv)
    gather_sum(ua)

    pltpu.sync_copy(pos_h.at[pl.ds(base, PB)], idxv)
    gather_sum(pb)
    dots()
    pltpu.sync_copy(scv, pos_out.at[pl.ds(base, PB)])

    pltpu.sync_copy(neg_h.at[pl.ds(base, PB)], idxv)
    gather_sum(pb)
    dots()
    pltpu.sync_copy(scv, neg_out.at[pl.ds(base, PB)])


def kernel(user_emb, item_emb, edge_weight, edge_index, users, pos_items,
           neg_items):
    f32 = jnp.float32
    pad_rows = jnp.zeros((PAD, D), f32)
    emb0 = jnp.concatenate([user_emb, pad_rows, item_emb, pad_rows], axis=0)

    src = edge_index[0]
    dst = edge_index[1]
    src_adj = src + PAD * (src >= N_USERS).astype(jnp.int32)
    epad = EP - E
    src_p = jnp.concatenate([src_adj, jnp.zeros((epad,), jnp.int32)])
    dst_p = jnp.concatenate([dst, jnp.zeros((epad,), jnp.int32)])
    w_p = jnp.concatenate([edge_weight, jnp.zeros((epad,), f32)])

    src2 = src_p.reshape(NROWS_IN, SUB)
    dst2 = dst_p.reshape(NROWS_IN, SUB)
    w2 = lax.bitcast_convert_type(w_p, jnp.int32).reshape(NROWS_IN, SUB)
    zeros = jnp.zeros((TROWS, D), f32)

    srcr, dstr, wr, cnts = _partition(src2, dst2, w2)

    x = emb0
    tables = [emb0]
    for _ in range(3):
        x = _layer(x, srcr, dstr, wr, cnts, zeros)
        tables.append(x)

    pos_s, neg_s = _score(
        tables[0], tables[1], tables[2], tables[3],
        users,
        pos_items + HP,
        neg_items + HP,
    )
    return (pos_s, neg_s)


# speedup vs baseline: 1.1155x; 1.1155x over previous
import functools

import jax
import jax.numpy as jnp
from jax import lax
from jax.experimental import pallas as pl
from jax.experimental.pallas import tpu as pltpu
from jax.experimental.pallas import tpu_sc as plsc

N_USERS = 25000
N_NODES = 50000
D = 64
E = 800000
B = 4096

NC = 2
NS = 16
NW = NC * NS
HALF = N_NODES // NC
TROWS = 1568
HP = NS * TROWS
NP = NC * HP
PAD = HP - HALF

SUB = 112
NSUB = 2
CHUNK = SUB * NSUB
EP = 802816
NROWS_IN = EP // SUB
WROWS = NROWS_IN // NW
NREG = 2 * NW
RROWS = WROWS
STG = 240

PB = B // NW

_mesh = plsc.VectorSubcoreMesh(
    core_axis_name="c", subcore_axis_name="s", num_cores=NC, num_subcores=NS)

_params = pltpu.CompilerParams(
    use_tc_tiling_on_sc=False, needs_layout_passes=False)


@functools.partial(
    pl.kernel,
    out_type=(jax.ShapeDtypeStruct((NREG * RROWS, SUB), jnp.int32),
              jax.ShapeDtypeStruct((NREG * RROWS, SUB), jnp.int32),
              jax.ShapeDtypeStruct((NREG * RROWS, SUB), jnp.int32),
              jax.ShapeDtypeStruct((NREG * 16,), jnp.int32)),
    mesh=_mesh,
    compiler_params=_params,
    scratch_types=[
        pltpu.VMEM((NSUB, SUB), jnp.int32),
        pltpu.VMEM((NSUB, SUB), jnp.int32),
        pltpu.VMEM((NSUB, SUB), jnp.int32),
        pltpu.VMEM((NSUB, SUB), jnp.int32),
        pltpu.VMEM((NSUB, SUB), jnp.int32),
        pltpu.VMEM((NSUB, SUB), jnp.int32),
        pltpu.SemaphoreType.DMA,
        pltpu.SemaphoreType.DMA,
        pltpu.VMEM((1, STG), jnp.int32),
        pltpu.VMEM((1, STG), jnp.int32),
        pltpu.VMEM((1, STG), jnp.int32),
        pltpu.VMEM((1, STG), jnp.int32),
        pltpu.VMEM((1, STG), jnp.int32),
        pltpu.VMEM((1, STG), jnp.int32),
        pltpu.VMEM((16,), jnp.int32),
    ],
)
def _partition(src_in, dst_in, w_in, src_out, dst_out, w_out, cnt_out,
               ins0, ind0, inw0, ins1, ind1, inw1, semi0, semi1,
               sa, da, wa, sb, db, wb, cntv):
    c = lax.axis_index("c")
    s = lax.axis_index("s")
    w = s * NC + c
    in0 = w * WROWS
    rb_a = (w * 2 + 0) * RROWS
    rb_b = (w * 2 + 1) * RROWS

    zero16 = jnp.zeros((16,), jnp.int32)
    iota16 = lax.iota(jnp.int32, 16)

    for buf in (sa, da, wa, sb, db, wb):
        for g in range(STG // 16):
            buf[0, pl.ds(g * 16, 16)] = zero16

    def flush(stg3, rb, orow):
        for stg, outp in zip(stg3, (src_out, dst_out, w_out)):
            pltpu.sync_copy(stg.at[pl.ds(0, 1), pl.ds(0, SUB)],
                            outp.at[pl.ds(rb + orow, 1)])
        for stg in stg3:
            for g in range((STG - SUB) // 16):
                stg[0, pl.ds(g * 16, 16)] = stg[0, pl.ds(SUB + g * 16, 16)]

    KIN = WROWS // NSUB
    insets = ((ins0, ind0, inw0, semi0), (ins1, ind1, inw1, semi1))

    def issue_in(k, st):
        ins, ind, inw, semi = st
        row = in0 + k * NSUB
        pltpu.async_copy(src_in.at[pl.ds(row, NSUB)], ins, semi)
        pltpu.async_copy(dst_in.at[pl.ds(row, NSUB)], ind, semi)
        pltpu.async_copy(w_in.at[pl.ds(row, NSUB)], inw, semi)

    def drain_in(st):
        ins, ind, inw, semi = st
        for buf in (ins, ind, inw):
            pltpu.make_async_copy(src_in.at[pl.ds(0, NSUB)], buf, semi).wait()

    def chunk(k, carry, st, nxt_st):
        pos_a, orow_a, pos_b, orow_b = carry
        ins, ind, inw, _ = st
        drain_in(st)

        @pl.when(k + 1 < KIN)
        def _():
            issue_in(k + 1, nxt_st)

        for r in range(NSUB):
            for g in range(SUB // 16):
                sl = pl.ds(g * 16, 16)
                s16 = ins[r, sl]
                d16 = ind[r, sl]
                w16 = inw[r, sl]
                m_a = d16 < HALF
                plsc.store_compressed(sa.at[0, pl.ds(pos_a, 16)], s16, mask=m_a)
                plsc.store_compressed(da.at[0, pl.ds(pos_a, 16)], d16, mask=m_a)
                plsc.store_compressed(wa.at[0, pl.ds(pos_a, 16)], w16, mask=m_a)
                m_b = jnp.logical_not(m_a)
                d16b = d16 - HALF
                plsc.store_compressed(sb.at[0, pl.ds(pos_b, 16)], s16, mask=m_b)
                plsc.store_compressed(db.at[0, pl.ds(pos_b, 16)], d16b, mask=m_b)
                plsc.store_compressed(wb.at[0, pl.ds(pos_b, 16)], w16, mask=m_b)
                n_a = lax.reduce_max(
                    plsc.all_reduce_population_count(m_a), axes=(0,))
                pos_a = pos_a + n_a
                pos_b = pos_b + (16 - n_a)
            do_a = pos_a >= SUB

            @pl.when(do_a)
            def _():
                flush((sa, da, wa), rb_a, orow_a)

            pos_a = jnp.where(do_a, pos_a - SUB, pos_a)
            orow_a = jnp.where(do_a, orow_a + 1, orow_a)

            do_b = pos_b >= SUB

            @pl.when(do_b)
            def _():
                flush((sb, db, wb), rb_b, orow_b)

            pos_b = jnp.where(do_b, pos_b - SUB, pos_b)
            orow_b = jnp.where(do_b, orow_b + 1, orow_b)
        return pos_a, orow_a, pos_b, orow_b

    z = jnp.int32(0)
    issue_in(0, insets[0])

    def chunk_pair(kk, carry):
        carry = chunk(2 * kk, carry, insets[0], insets[1])
        carry = chunk(2 * kk + 1, carry, insets[1], insets[0])
        return carry

    pos_a, orow_a, pos_b, orow_b = lax.fori_loop(
        0, KIN // 2, chunk_pair, (z, z, z, z))

    def finish(stg3, rb, pos, orow):
        stg_w = stg3[2]
        for g in range(SUB // 16):
            sl = pl.ds(g * 16, 16)
            keep = (iota16 + g * 16) < pos
            stg_w[0, sl] = jnp.where(keep, stg_w[0, sl], 0)

        @pl.when(pos > 0)
        def _():
            flush(stg3, rb, orow)

        orow = jnp.where(pos > 0, orow + 1, orow)
        for g in range(STG // 16):
            stg_w[0, pl.ds(g * 16, 16)] = zero16
        for _ in range(3):
            do = lax.rem(orow, jnp.int32(4)) != 0

            @pl.when(do)
            def _():
                flush(stg3, rb, orow)

            orow = jnp.where(do, orow + 1, orow)
        return orow

    orow_a = finish((sa, da, wa), rb_a, pos_a, orow_a)
    orow_b = finish((sb, db, wb), rb_b, pos_b, orow_b)

    cntv[pl.ds(0, 16)] = jnp.full((16,), 1, jnp.int32) * (orow_a // NSUB)
    pltpu.sync_copy(cntv, cnt_out.at[pl.ds((w * 2 + 0) * 16, 16)])
    cntv[pl.ds(0, 16)] = jnp.full((16,), 1, jnp.int32) * (orow_b // NSUB)
    pltpu.sync_copy(cntv, cnt_out.at[pl.ds((w * 2 + 1) * 16, 16)])


@functools.partial(
    pl.kernel,
    out_type=jax.ShapeDtypeStruct((NP, D), jnp.float32),
    mesh=_mesh,
    compiler_params=_params,
    scratch_types=[
        pltpu.VMEM((NSUB, SUB), jnp.int32),
        pltpu.VMEM((NSUB, SUB), jnp.int32),
        pltpu.VMEM((NSUB, SUB), jnp.int32),
        pltpu.VMEM((NSUB, SUB), jnp.int32),
        pltpu.VMEM((NSUB, SUB), jnp.int32),
        pltpu.VMEM((NSUB, SUB), jnp.int32),
        pltpu.VMEM((CHUNK, D), jnp.float32),
        pltpu.VMEM((CHUNK, D), jnp.float32),
        pltpu.VMEM((NSUB, SUB), jnp.int32),
        pltpu.VMEM((NSUB, SUB), jnp.int32),
        pltpu.VMEM((16,), jnp.int32),
        pltpu.VMEM_SHARED((HP, D), jnp.float32),
        pltpu.SemaphoreType.DMA,
        pltpu.SemaphoreType.DMA,
        pltpu.SemaphoreType.DMA,
        pltpu.SemaphoreType.DMA,
        pltpu.SemaphoreType.DMA,
        pltpu.SemaphoreType.DMA,
    ],
)
def _layer(emb, src_p, dst_p, w_p, cnt_p, zeros_hbm, out,
           ea, da, wa, eb, db, wb, ra, rb_, dca, dcb, cntv, acc,
           semea, semeb, semga, semgb, semsa, semsb):
    c = lax.axis_index("c")
    s = lax.axis_index("s")
    r0 = s * TROWS

    sets = ((ea, da, wa, ra, dca, semea, semga, semsa),
            (eb, db, wb, rb_, dcb, semeb, semgb, semsb))

    pltpu.sync_copy(zeros_hbm, acc.at[pl.ds(r0, TROWS)])
    plsc.subcore_barrier()

    def region(i):
        reg = (2 * s + i) * 2 + c
        rbase = reg * RROWS

        pltpu.sync_copy(cnt_p.at[pl.ds(reg * 16, 16)], cntv)
        kd = lax.reduce_max(cntv[pl.ds(0, 16)], axes=(0,))

        def issue_edata(k, st):
            es, ds_, ws, _, _, seme, _, _ = st
            row = rbase + k * NSUB
            pltpu.async_copy(src_p.at[pl.ds(row, NSUB)], es, seme)
            pltpu.async_copy(dst_p.at[pl.ds(row, NSUB)], ds_, seme)
            pltpu.async_copy(w_p.at[pl.ds(row, NSUB)], ws, seme)

        def drain_edata(st):
            es, ds_, ws, _, _, seme, _, _ = st
            for buf in (es, ds_, ws):
                pltpu.make_async_copy(
                    src_p.at[pl.ds(0, NSUB)], buf, seme).wait()

        def issue_gather(st):
            es, _, _, r, _, _, semg, _ = st
            for j in range(NSUB):
                pltpu.async_copy(emb.at[es.at[j]],
                                 r.at[pl.ds(j * SUB, SUB)], semg)

        def drain_rows(st, which):
            _, _, _, r, _, _, semg, sems = st
            sem = semg if which == "g" else sems
            pltpu.make_async_copy(out.at[pl.ds(0, CHUNK)], r, sem).wait()

        def issue_scatter(st):
            _, _, _, r, dc, _, _, sems = st
            for j in range(NSUB):
                pltpu.async_copy(r.at[pl.ds(j * SUB, SUB)],
                                 acc.at[dc.at[j]], sems, add=True)

        def compute(st):
            _, ds_, ws, r, dc, _, _, _ = st

            for jj in range(NSUB):
                for g in range(SUB // 16):
                    sl = pl.ds(g * 16, 16)
                    dc[jj, sl] = ds_[jj, sl]

            def row_body(jj, rc):
                for cc in range(SUB // 16):
                    sl = pl.ds(cc * 16, 16)
                    w16 = plsc.bitcast(ws[jj, sl], jnp.float32)
                    e0 = jj * SUB + cc * 16
                    for i2 in range(16):
                        wsp = w16.at[jnp.full((16,), i2, jnp.int32)].get(
                            mode="promise_in_bounds")
                        for j in range(D // 16):
                            sj = pl.ds(j * 16, 16)
                            r[e0 + i2, sj] = r[e0 + i2, sj] * wsp
                return rc

            lax.fori_loop(0, NSUB, row_body, 0)

        @pl.when(kd > 0)
        def _():
            issue_edata(0, sets[0])
            drain_edata(sets[0])
            issue_gather(sets[0])

        @pl.when(kd > 1)
        def _():
            issue_edata(1, sets[1])

        def step(k, cur, nxt):
            @pl.when(k + 1 < kd)
            def _():
                drain_edata(nxt)

            @pl.when(k >= 1)
            def _():
                drain_rows(nxt, "s")

            @pl.when(k + 1 < kd)
            def _():
                issue_gather(nxt)

            drain_rows(cur, "g")
            compute(cur)
            issue_scatter(cur)

            @pl.when(k + 2 < kd)
            def _():
                issue_edata(k + 2, cur)

        def chunk_pair(kk, carry):
            step(2 * kk, sets[0], sets[1])
            step(2 * kk + 1, sets[1], sets[0])
            return carry

        lax.fori_loop(0, kd // 2, chunk_pair, 0)

        @pl.when(kd > 0)
        def _():
            drain_rows(sets[1], "s")

    region(0)
    region(1)
    plsc.subcore_barrier()

    pltpu.sync_copy(acc.at[pl.ds(r0, TROWS)],
                    out.at[pl.ds(c * HP + r0, TROWS)])


@functools.partial(
    pl.kernel,
    out_type=(jax.ShapeDtypeStruct((B,), jnp.float32),
              jax.ShapeDtypeStruct((B,), jnp.float32)),
    mesh=_mesh,
    compiler_params=_params,
    scratch_types=[
        pltpu.VMEM((PB,), jnp.int32),
        pltpu.VMEM((PB, D), jnp.float32),
        pltpu.VMEM((PB, D), jnp.float32),
        pltpu.VMEM((PB, D), jnp.float32),
        pltpu.VMEM((PB,), jnp.float32),
        pltpu.SemaphoreType.DMA,
    ],
)
def _score(t0, t1, t2, t3, users_h, pos_h, neg_h, pos_out, neg_out,
           idxv, ua, pb, tmp, scv, sem):
    c = lax.axis_index("c")
    s = lax.axis_index("s")
    wid = s * NC + c
    base = wid * PB

    def gather_sum(dst_buf):
        pltpu.async_copy(t0.at[idxv], dst_buf, sem).wait()
        for t in (t1, t2, t3):
            pltpu.async_copy(t.at[idxv], tmp, sem).wait()

            def add_body(r, rc):
                for j in range(D // 16):
                    sj = pl.ds(j * 16, 16)
                    dst_buf[r, sj] = dst_buf[r, sj] + tmp[r, sj]
                return rc

            lax.fori_loop(0, PB, add_body, 0)

    lane0 = lax.iota(jnp.int32, 16) == 0

    def dots():
        def dot_body(e, rc):
            acc16 = ua[e, pl.ds(0, 16)] * pb[e, pl.ds(0, 16)]
            for j in range(1, D // 16):
                sj = pl.ds(j * 16, 16)
                acc16 = acc16 + ua[e, sj] * pb[e, sj]
            sc = jnp.sum(acc16) * jnp.float32(1.0 / 16.0)
            plsc.store_scatter(scv, [jnp.full((16,), e, jnp.int32)],
                               jnp.full((16,), sc, jnp.float32), mask=lane0)
            return rc

        lax.fori_loop(0, PB, dot_body, 0)

    pltpu.sync_copy(users_h.at[pl.ds(base, PB)], idxv)
    gather_sum(ua)

    pltpu.sync_copy(pos_h.at[pl.ds(base, PB)], idxv)
    gather_sum(pb)
    dots()
    pltpu.sync_copy(scv, pos_out.at[pl.ds(base, PB)])

    pltpu.sync_copy(neg_h.at[pl.ds(base, PB)], idxv)
    gather_sum(pb)
    dots()
    pltpu.sync_copy(scv, neg_out.at[pl.ds(base, PB)])


def kernel(user_emb, item_emb, edge_weight, edge_index, users, pos_items,
           neg_items):
    f32 = jnp.float32
    pad_rows = jnp.zeros((PAD, D), f32)
    emb0 = jnp.concatenate([user_emb, pad_rows, item_emb, pad_rows], axis=0)

    src = edge_index[0]
    dst = edge_index[1]
    src_adj = src + PAD * (src >= N_USERS).astype(jnp.int32)
    epad = EP - E
    src_p = jnp.concatenate([src_adj, jnp.zeros((epad,), jnp.int32)])
    dst_p = jnp.concatenate([dst, jnp.zeros((epad,), jnp.int32)])
    w_p = jnp.concatenate([edge_weight, jnp.zeros((epad,), f32)])

    src2 = src_p.reshape(NROWS_IN, SUB)
    dst2 = dst_p.reshape(NROWS_IN, SUB)
    w2 = lax.bitcast_convert_type(w_p, jnp.int32).reshape(NROWS_IN, SUB)
    zeros = jnp.zeros((TROWS, D), f32)

    srcr, dstr, wr, cnts = _partition(src2, dst2, w2)

    x = emb0
    tables = [emb0]
    for _ in range(3):
        x = _layer(x, srcr, dstr, wr, cnts, zeros)
        tables.append(x)

    pos_s, neg_s = _score(
        tables[0], tables[1], tables[2], tables[3],
        users,
        pos_items + HP,
        neg_items + HP,
    )
    return (pos_s, neg_s)
